# baseline (speedup 1.0000x reference)
import functools

import jax
import jax.numpy as jnp
from jax import lax
from jax.experimental import pallas as pl
from jax.experimental.pallas import tpu as pltpu
from jax.experimental.pallas import tpu_sc as plsc

BATCH = 16384
EMBED_DIM = 64
NUM_CORES = 2
NUM_SUBCORES = 16
LANES = 16
NUM_WORKERS = NUM_CORES * NUM_SUBCORES
B_PER_W = BATCH // NUM_WORKERS

_mesh = plsc.VectorSubcoreMesh(core_axis_name="c", subcore_axis_name="s")


@functools.partial(
    pl.kernel,
    mesh=_mesh,
    out_type=jax.ShapeDtypeStruct((BATCH,), jnp.float32),
    scratch_types=[
        pltpu.VMEM((B_PER_W,), jnp.int32),
        pltpu.VMEM((B_PER_W,), jnp.int32),
        pltpu.VMEM((B_PER_W, EMBED_DIM), jnp.float32),
        pltpu.VMEM((B_PER_W, EMBED_DIM), jnp.float32),
        pltpu.VMEM((B_PER_W,), jnp.float32),
        pltpu.SemaphoreType.DMA,
        pltpu.SemaphoreType.DMA,
    ],
    compiler_params=pltpu.CompilerParams(
        needs_layout_passes=False, use_tc_tiling_on_sc=False),
)
def _sc_dot(user_hbm, item_hbm, utab_hbm, itab_hbm, out_hbm,
            uidx_v, iidx_v, urows_v, irows_v, out_v, sem_u, sem_i):
    wid = lax.axis_index("s") * NUM_CORES + lax.axis_index("c")
    base = wid * B_PER_W

    pltpu.sync_copy(user_hbm.at[pl.ds(base, B_PER_W)], uidx_v)
    pltpu.sync_copy(item_hbm.at[pl.ds(base, B_PER_W)], iidx_v)

    cu = pltpu.async_copy(utab_hbm.at[uidx_v], urows_v, sem_u)
    ci = pltpu.async_copy(itab_hbm.at[iidx_v], irows_v, sem_i)
    cu.wait()
    ci.wait()

    lane_iota = lax.iota(jnp.int32, LANES)

    def group_body(g, _):
        rows = g * LANES + lane_iota
        acc = jnp.zeros((LANES,), jnp.float32)
        for d in range(EMBED_DIM):
            col = jnp.full((LANES,), d, jnp.int32)
            u = plsc.load_gather(urows_v, [rows, col])
            v = plsc.load_gather(irows_v, [rows, col])
            acc = acc + u * v
        out_v[pl.ds(pl.multiple_of(g * LANES, LANES), LANES)] = acc
        return 0

    lax.fori_loop(0, B_PER_W // LANES, group_body, 0)

    pltpu.sync_copy(out_v, out_hbm.at[pl.ds(base, B_PER_W)])


def kernel(user, item, user_table, item_table):
    return _sc_dot(user, item, user_table, item_table)


# speedup vs baseline: 1.5609x; 1.5609x over previous
import functools

import jax
import jax.numpy as jnp
from jax import lax
from jax.experimental import pallas as pl
from jax.experimental.pallas import tpu as pltpu
from jax.experimental.pallas import tpu_sc as plsc

BATCH = 16384
EMBED_DIM = 64
NUM_CORES = 2
NUM_SUBCORES = 16
LANES = 16
NUM_WORKERS = NUM_CORES * NUM_SUBCORES
B_PER_W = BATCH // NUM_WORKERS
CHUNK = 256
N_CHUNKS = B_PER_W // CHUNK

_mesh = plsc.VectorSubcoreMesh(core_axis_name="c", subcore_axis_name="s")


@functools.partial(
    pl.kernel,
    mesh=_mesh,
    out_type=jax.ShapeDtypeStruct((BATCH,), jnp.float32),
    scratch_types=[
        pltpu.VMEM((B_PER_W,), jnp.int32),
        pltpu.VMEM((B_PER_W,), jnp.int32),
        pltpu.VMEM((CHUNK, EMBED_DIM), jnp.float32),
        pltpu.VMEM((CHUNK, EMBED_DIM), jnp.float32),
        pltpu.VMEM((B_PER_W,), jnp.float32),
        pltpu.SemaphoreType.DMA,
        pltpu.SemaphoreType.DMA,
    ],
    compiler_params=pltpu.CompilerParams(
        needs_layout_passes=False, use_tc_tiling_on_sc=True),
)
def _sc_dot(user_hbm, item_hbm, utab_hbm, itab_hbm, out_hbm,
            uidx_v, iidx_v, urows_v, irows_v, out_v, sem_u, sem_i):
    wid = lax.axis_index("s") * NUM_CORES + lax.axis_index("c")
    base = wid * B_PER_W

    pltpu.sync_copy(user_hbm.at[pl.ds(base, B_PER_W)], uidx_v)
    pltpu.sync_copy(item_hbm.at[pl.ds(base, B_PER_W)], iidx_v)

    lane_iota = lax.iota(jnp.int32, LANES)

    def chunk_body(c, _):
        coff = c * CHUNK

        def fire(g, _):
            goff = pl.multiple_of(coff + g * LANES, LANES)
            uvec = uidx_v[pl.ds(goff, LANES)]
            ivec = iidx_v[pl.ds(goff, LANES)]
            for k in range(LANES):
                ur = uvec[k]
                ir = ivec[k]
                pltpu.async_copy(utab_hbm.at[ur],
                                 urows_v.at[g * LANES + k], sem_u)
                pltpu.async_copy(itab_hbm.at[ir],
                                 irows_v.at[g * LANES + k], sem_i)
            return 0

        lax.fori_loop(0, CHUNK // LANES, fire, 0)

        def drain(r, _):
            pltpu.make_async_copy(utab_hbm.at[0], urows_v.at[0], sem_u).wait()
            pltpu.make_async_copy(itab_hbm.at[0], irows_v.at[0], sem_i).wait()
            return 0

        lax.fori_loop(0, CHUNK, drain, 0)

        def group_body(g, _):
            rows = g * LANES + lane_iota
            acc = jnp.zeros((LANES,), jnp.float32)
            for d in range(EMBED_DIM):
                col = jnp.full((LANES,), d, jnp.int32)
                u = plsc.load_gather(urows_v, [rows, col])
                v = plsc.load_gather(irows_v, [rows, col])
                acc = acc + u * v
            out_v[pl.ds(pl.multiple_of(coff + g * LANES, LANES), LANES)] = acc
            return 0

        lax.fori_loop(0, CHUNK // LANES, group_body, 0)
        return 0

    lax.fori_loop(0, N_CHUNKS, chunk_body, 0)

    pltpu.sync_copy(out_v, out_hbm.at[pl.ds(base, B_PER_W)])


def kernel(user, item, user_table, item_table):
    return _sc_dot(user, item, user_table, item_table)


# speedup vs baseline: 1.6228x; 1.0397x over previous
import functools

import jax
import jax.numpy as jnp
from jax import lax
from jax.experimental import pallas as pl
from jax.experimental.pallas import tpu as pltpu
from jax.experimental.pallas import tpu_sc as plsc

BATCH = 16384
EMBED_DIM = 64
NUM_CORES = 2
NUM_SUBCORES = 16
LANES = 16
NUM_WORKERS = NUM_CORES * NUM_SUBCORES
B_PER_W = BATCH // NUM_WORKERS
CHUNK = 256
N_CHUNKS = B_PER_W // CHUNK

_mesh = plsc.VectorSubcoreMesh(core_axis_name="c", subcore_axis_name="s")


@functools.partial(
    pl.kernel,
    mesh=_mesh,
    out_type=jax.ShapeDtypeStruct((BATCH,), jnp.float32),
    scratch_types=[
        pltpu.VMEM((B_PER_W,), jnp.int32),
        pltpu.VMEM((B_PER_W,), jnp.int32),
        pltpu.VMEM((CHUNK, EMBED_DIM), jnp.float32),
        pltpu.VMEM((CHUNK, EMBED_DIM), jnp.float32),
        pltpu.VMEM((B_PER_W,), jnp.float32),
        pltpu.SemaphoreType.DMA,
        pltpu.SemaphoreType.DMA,
    ],
    compiler_params=pltpu.CompilerParams(
        needs_layout_passes=False, use_tc_tiling_on_sc=True),
)
def _sc_dot(user_hbm, item_hbm, utab_hbm, itab_hbm, out_hbm,
            uidx_v, iidx_v, urows_v, irows_v, out_v, sem_u, sem_i):
    wid = lax.axis_index("s") * NUM_CORES + lax.axis_index("c")
    base = wid * B_PER_W

    pltpu.sync_copy(user_hbm.at[pl.ds(base, B_PER_W)], uidx_v)
    pltpu.sync_copy(item_hbm.at[pl.ds(base, B_PER_W)], iidx_v)

    lane_iota = lax.iota(jnp.int32, LANES)

    def chunk_body(c, _):
        coff = c * CHUNK

        def fire(g, _):
            goff = pl.multiple_of(coff + g * LANES, LANES)
            uvec = uidx_v[pl.ds(goff, LANES)]
            ivec = iidx_v[pl.ds(goff, LANES)]
            for k in range(LANES):
                ur = uvec[k]
                ir = ivec[k]
                pltpu.async_copy(utab_hbm.at[ur],
                                 urows_v.at[g * LANES + k], sem_u)
                pltpu.async_copy(itab_hbm.at[ir],
                                 irows_v.at[g * LANES + k], sem_i)
            return 0

        lax.fori_loop(0, CHUNK // LANES, fire, 0)

        def drain(r, _):
            pltpu.make_async_copy(utab_hbm.at[0], urows_v.at[0], sem_u).wait()
            pltpu.make_async_copy(itab_hbm.at[0], irows_v.at[0], sem_i).wait()
            return 0

        lax.fori_loop(0, CHUNK, drain, 0)

        def group_body(g, _):
            acc = jnp.zeros((LANES,), jnp.float32)
            for k in range(LANES):
                r = g * LANES + k
                s = None
                for cc in range(0, EMBED_DIM, LANES):
                    u = urows_v[r, pl.ds(cc, LANES)]
                    v = irows_v[r, pl.ds(cc, LANES)]
                    s = u * v if s is None else s + u * v
                total = jnp.sum(s)
                acc = jnp.where(lane_iota == k, total, acc)
            out_v[pl.ds(pl.multiple_of(coff + g * LANES, LANES), LANES)] = acc
            return 0

        lax.fori_loop(0, CHUNK // LANES, group_body, 0)
        return 0

    lax.fori_loop(0, N_CHUNKS, chunk_body, 0)

    pltpu.sync_copy(out_v, out_hbm.at[pl.ds(base, B_PER_W)])


def kernel(user, item, user_table, item_table):
    return _sc_dot(user, item, user_table, item_table)
